# baseline (speedup 1.0000x reference)
import functools

import jax
import jax.numpy as jnp
from jax import lax
from jax.experimental import pallas as pl
from jax.experimental.pallas import tpu as pltpu
from jax.experimental.pallas import tpu_sc as plsc

_K = 9
_NBR = _K - 1
_TQ = 512
_TR = 512
_CP = 128
_NCHUNKQ = 2


def _table_body(qt_ref, tab_ref):
    qt = qt_ref[0]
    tab_ref[0] = jnp.concatenate(
        [qt.T, jnp.zeros((qt.shape[1], _CP - qt.shape[0]), jnp.float32)],
        axis=1)


def _table_call(xf):
    b, c, n = xf.shape
    return pl.pallas_call(
        _table_body,
        grid=(b, n // _TQ),
        in_specs=[pl.BlockSpec((1, c, _TQ), lambda i, q: (i, 0, q))],
        out_specs=pl.BlockSpec((1, _TQ, _CP), lambda i, q: (i, q, 0)),
        out_shape=jax.ShapeDtypeStruct((b, n, _CP), jnp.float32),
    )(xf)


def _topk_body(qt_ref, xf_ref, idx_ref, *, q0):
    qi = pl.program_id(0)
    qt = qt_ref[0]
    xf = xf_ref[0]
    c, n = xf.shape
    dot = lax.dot_general(qt, xf, (((0,), (0,)), ((), ())),
                          preferred_element_type=jnp.float32)
    nk = jnp.sum(xf * xf, axis=0, keepdims=True)
    r = nk - 2.0 * dot
    lane = lax.broadcasted_iota(jnp.int32, r.shape, 1)
    row = q0 + qi * _TQ + lax.broadcasted_iota(jnp.int32, r.shape, 0)
    inf = jnp.float32(jnp.inf)
    r = jnp.where(lane == row, inf, r)
    lane_f = lane.astype(jnp.float32)
    lane8 = lax.broadcasted_iota(jnp.int32, (r.shape[0], _NBR), 1)
    acc = jnp.zeros((r.shape[0], _NBR), jnp.float32)
    for k in range(_NBR):
        m = jnp.min(r, axis=1, keepdims=True)
        i_f = jnp.min(jnp.where(r == m, lane_f, inf), axis=1,
                      keepdims=True)
        acc = jnp.where(lane8 == k, i_f, acc)
        r = jnp.where(lane_f == i_f, inf, r)
    idx_ref[...] = acc.astype(jnp.int32)


def _topk_call(xf, b, q0, nq):
    _, c, n = xf.shape
    t0 = q0 // _TQ
    return pl.pallas_call(
        functools.partial(_topk_body, q0=q0),
        grid=(nq // _TQ,),
        in_specs=[
            pl.BlockSpec((1, c, _TQ), lambda q: (b, 0, t0 + q)),
            pl.BlockSpec((1, c, n), lambda q: (b, 0, 0)),
        ],
        out_specs=pl.BlockSpec((_TQ, _NBR), lambda q: (q, 0)),
        out_shape=jax.ShapeDtypeStruct((nq, _NBR), jnp.int32),
    )(xf, xf)


def _make_gather(tot, cp):
    info = plsc.get_sparse_core_info()
    nw = info.num_cores * info.num_subcores
    per_w = tot // nw
    nchunk = per_w // 128
    nbuf = min(6, nchunk)
    ahead = max(1, nbuf - 2)
    mesh = plsc.VectorSubcoreMesh(core_axis_name="c", subcore_axis_name="s")

    @functools.partial(
        pl.kernel, mesh=mesh,
        out_type=jax.ShapeDtypeStruct((tot, cp), jnp.float32),
        scratch_types=[
            pltpu.VMEM((nchunk, 128), jnp.int32),
            pltpu.VMEM((nbuf, 128, cp), jnp.float32),
            pltpu.SemaphoreType.DMA,
            pltpu.SemaphoreType.DMA,
        ],
    )
    def gath(table_hbm, idx_hbm, out_hbm, idx_v, rows_v, gsem, wsem):
        wid = lax.axis_index("s") * info.num_cores + lax.axis_index("c")
        base = wid * per_w
        pltpu.sync_copy(idx_hbm.at[pl.ds(wid * nchunk, nchunk)], idx_v)
        gd = [None] * nchunk
        wd = [None] * nchunk
        for ck in range(min(ahead, nchunk)):
            gd[ck] = pltpu.async_copy(
                table_hbm.at[idx_v.at[ck]], rows_v.at[ck % nbuf], gsem)
        for ck in range(nchunk):
            nx = ck + ahead
            if nx < nchunk:
                if nx >= nbuf:
                    wd[nx - nbuf].wait()
                gd[nx] = pltpu.async_copy(
                    table_hbm.at[idx_v.at[nx]], rows_v.at[nx % nbuf], gsem)
            gd[ck].wait()
            wd[ck] = pltpu.async_copy(
                rows_v.at[ck % nbuf],
                out_hbm.at[pl.ds(base + ck * 128, 128)], wsem)
        for ck in range(max(0, nchunk - nbuf), nchunk):
            wd[ck].wait()

    return gath


def _conv_body(xfq_ref, g_ref, w0_ref, wr_ref, bias_ref, out_ref, acc_ref):
    k = pl.program_id(1)
    c = w0_ref.shape[0]
    part = lax.dot_general(g_ref[0][:, :c], wr_ref[0],
                           (((1,), (0,)), ((), ())),
                           preferred_element_type=jnp.float32)

    @pl.when(k == 0)
    def _init():
        acc_ref[...] = part + lax.dot_general(
            xfq_ref[0], w0_ref[...], (((0,), (0,)), ((), ())),
            preferred_element_type=jnp.float32) + bias_ref[...]

    @pl.when(k > 0)
    def _accum():
        acc_ref[...] += part

    @pl.when(k == _NBR - 1)
    def _flush():
        out_ref[...] = acc_ref[...].T


def _conv_call(xf, g3, w0, wr3, bias, b, q0):
    _, c, n = xf.shape
    nq = g3.shape[1]
    co = w0.shape[1]
    t0 = q0 // _TR
    return pl.pallas_call(
        _conv_body,
        grid=(nq // _TR, _NBR),
        in_specs=[
            pl.BlockSpec((1, c, _TR), lambda t, k: (b, 0, t0 + t)),
            pl.BlockSpec((1, _TR, _CP), lambda t, k: (k, t, 0)),
            pl.BlockSpec((c, co), lambda t, k: (0, 0)),
            pl.BlockSpec((1, c, co), lambda t, k: (k, 0, 0)),
            pl.BlockSpec((1, co), lambda t, k: (0, 0)),
        ],
        out_specs=pl.BlockSpec((co, _TR), lambda t, k: (0, t)),
        out_shape=jax.ShapeDtypeStruct((co, nq), jnp.float32),
        scratch_shapes=[pltpu.VMEM((_TR, co), jnp.float32)],
    )(xf, g3, w0, wr3, bias)


def kernel(x, conv_w, conv_b):
    b, c, h, w = x.shape
    n = h * w
    nq = n // _NCHUNKQ
    co = conv_w.shape[0]
    xf = x.reshape(b, c, n)

    wt = jnp.transpose(conv_w, (2, 1, 0))
    w0 = wt[0]
    wr3 = wt[1:]
    bias = conv_b.reshape(1, co)

    tables = _table_call(xf)
    gath = _make_gather(nq * _NBR, _CP)

    outs = []
    for bi in range(b):
        chunks = []
        for ci in range(_NCHUNKQ):
            q0 = ci * nq
            idx = _topk_call(xf, bi, q0, nq)
            flat_idx = idx.T.reshape(nq * _NBR // 128, 128)
            g = gath(tables[bi], flat_idx)
            g3 = g.reshape(_NBR, nq, _CP)
            chunks.append(_conv_call(xf, g3, w0, wr3, bias, bi, q0))
        outs.append(jnp.concatenate(chunks, axis=1))
    return jnp.stack(outs).reshape(b, co, h, w)


# speedup vs baseline: 1.0261x; 1.0261x over previous
import functools

import jax
import jax.numpy as jnp
from jax import lax
from jax.experimental import pallas as pl
from jax.experimental.pallas import tpu as pltpu
from jax.experimental.pallas import tpu_sc as plsc

_K = 9
_NBR = _K - 1
_TQ = 512
_TR = 512
_CP = 128
_NCHUNKQ = 2


def _table_body(qt_ref, tab_ref):
    qt = qt_ref[0]
    tab_ref[0] = jnp.concatenate(
        [qt.T, jnp.zeros((qt.shape[1], _CP - qt.shape[0]), jnp.float32)],
        axis=1)


def _table_call(xf):
    b, c, n = xf.shape
    return pl.pallas_call(
        _table_body,
        grid=(b, n // _TQ),
        in_specs=[pl.BlockSpec((1, c, _TQ), lambda i, q: (i, 0, q))],
        out_specs=pl.BlockSpec((1, _TQ, _CP), lambda i, q: (i, q, 0)),
        out_shape=jax.ShapeDtypeStruct((b, n, _CP), jnp.float32),
    )(xf)


def _topk_body(qt_ref, xf_ref, idx_ref, *, q0):
    qi = pl.program_id(0)
    qt = qt_ref[0]
    xf = xf_ref[0]
    c, n = xf.shape
    dot = lax.dot_general(qt, xf, (((0,), (0,)), ((), ())),
                          preferred_element_type=jnp.float32)
    nk = jnp.sum(xf * xf, axis=0, keepdims=True)
    r = nk - 2.0 * dot
    lane = lax.broadcasted_iota(jnp.int32, r.shape, 1)
    row = q0 + qi * _TQ + lax.broadcasted_iota(jnp.int32, r.shape, 0)
    inf = jnp.float32(jnp.inf)
    r = jnp.where(lane == row, inf, r)
    lane_f = lane.astype(jnp.float32)
    lane8 = lax.broadcasted_iota(jnp.int32, (r.shape[0], _NBR), 1)
    acc = jnp.zeros((r.shape[0], _NBR), jnp.float32)
    for k in range(_NBR):
        m = jnp.min(r, axis=1, keepdims=True)
        i_f = jnp.min(jnp.where(r == m, lane_f, inf), axis=1,
                      keepdims=True)
        acc = jnp.where(lane8 == k, i_f, acc)
        r = jnp.where(lane_f == i_f, inf, r)
    idx_ref[0] = acc.T.astype(jnp.int32)


def _topk_call(xf, b, q0, nq):
    _, c, n = xf.shape
    t0 = q0 // _TQ
    return pl.pallas_call(
        functools.partial(_topk_body, q0=q0),
        grid=(nq // _TQ,),
        in_specs=[
            pl.BlockSpec((1, c, _TQ), lambda q: (b, 0, t0 + q)),
            pl.BlockSpec((1, c, n), lambda q: (b, 0, 0)),
        ],
        out_specs=pl.BlockSpec((1, _NBR, _TQ), lambda q: (q, 0, 0)),
        out_shape=jax.ShapeDtypeStruct((nq // _TQ, _NBR, _TQ), jnp.int32),
    )(xf, xf)


def _make_gather(tot, cp):
    info = plsc.get_sparse_core_info()
    nw = info.num_cores * info.num_subcores
    per_w = tot // nw
    nchunk = per_w // 128
    nbuf = min(6, nchunk)
    ahead = max(1, nbuf - 2)
    mesh = plsc.VectorSubcoreMesh(core_axis_name="c", subcore_axis_name="s")

    @functools.partial(
        pl.kernel, mesh=mesh,
        out_type=jax.ShapeDtypeStruct((tot, cp), jnp.float32),
        scratch_types=[
            pltpu.VMEM((nchunk, 128), jnp.int32),
            pltpu.VMEM((nbuf, 128, cp), jnp.float32),
            pltpu.SemaphoreType.DMA,
            pltpu.SemaphoreType.DMA,
        ],
    )
    def gath(table_hbm, idx_hbm, out_hbm, idx_v, rows_v, gsem, wsem):
        wid = lax.axis_index("s") * info.num_cores + lax.axis_index("c")
        base = wid * per_w
        pltpu.sync_copy(idx_hbm.at[pl.ds(wid * nchunk, nchunk)], idx_v)
        gd = [None] * nchunk
        wd = [None] * nchunk
        for ck in range(min(ahead, nchunk)):
            gd[ck] = pltpu.async_copy(
                table_hbm.at[idx_v.at[ck]], rows_v.at[ck % nbuf], gsem)
        for ck in range(nchunk):
            nx = ck + ahead
            if nx < nchunk:
                if nx >= nbuf:
                    wd[nx - nbuf].wait()
                gd[nx] = pltpu.async_copy(
                    table_hbm.at[idx_v.at[nx]], rows_v.at[nx % nbuf], gsem)
            gd[ck].wait()
            wd[ck] = pltpu.async_copy(
                rows_v.at[ck % nbuf],
                out_hbm.at[pl.ds(base + ck * 128, 128)], wsem)
        for ck in range(max(0, nchunk - nbuf), nchunk):
            wd[ck].wait()

    return gath


def _conv_body(xfq_ref, g_ref, w0_ref, wr_ref, bias_ref, out_ref, acc_ref):
    k = pl.program_id(1)
    c = w0_ref.shape[0]
    part = lax.dot_general(g_ref[0][0][:, :c], wr_ref[0],
                           (((1,), (0,)), ((), ())),
                           preferred_element_type=jnp.float32)

    @pl.when(k == 0)
    def _init():
        acc_ref[...] = part + lax.dot_general(
            xfq_ref[0], w0_ref[...], (((0,), (0,)), ((), ())),
            preferred_element_type=jnp.float32) + bias_ref[...]

    @pl.when(k > 0)
    def _accum():
        acc_ref[...] += part

    @pl.when(k == _NBR - 1)
    def _flush():
        out_ref[...] = acc_ref[...].T


def _conv_call(xf, g4, w0, wr3, bias, b, q0):
    _, c, n = xf.shape
    nq = g4.shape[0] * g4.shape[2]
    co = w0.shape[1]
    t0 = q0 // _TR
    return pl.pallas_call(
        _conv_body,
        grid=(nq // _TR, _NBR),
        in_specs=[
            pl.BlockSpec((1, c, _TR), lambda t, k: (b, 0, t0 + t)),
            pl.BlockSpec((1, 1, _TR, _CP), lambda t, k: (t, k, 0, 0)),
            pl.BlockSpec((c, co), lambda t, k: (0, 0)),
            pl.BlockSpec((1, c, co), lambda t, k: (k, 0, 0)),
            pl.BlockSpec((1, co), lambda t, k: (0, 0)),
        ],
        out_specs=pl.BlockSpec((co, _TR), lambda t, k: (0, t)),
        out_shape=jax.ShapeDtypeStruct((co, nq), jnp.float32),
        scratch_shapes=[pltpu.VMEM((_TR, co), jnp.float32)],
    )(xf, g4, w0, wr3, bias)


def kernel(x, conv_w, conv_b):
    b, c, h, w = x.shape
    n = h * w
    nq = n // _NCHUNKQ
    co = conv_w.shape[0]
    xf = x.reshape(b, c, n)

    wt = jnp.transpose(conv_w, (2, 1, 0))
    w0 = wt[0]
    wr3 = wt[1:]
    bias = conv_b.reshape(1, co)

    tables = _table_call(xf)
    gath = _make_gather(nq * _NBR, _CP)

    outs = []
    for bi in range(b):
        chunks = []
        for ci in range(_NCHUNKQ):
            q0 = ci * nq
            idx = _topk_call(xf, bi, q0, nq)
            flat_idx = idx.reshape(nq * _NBR // 128, 128)
            g = gath(tables[bi], flat_idx)
            g4 = g.reshape(nq // _TQ, _NBR, _TQ, _CP)
            chunks.append(_conv_call(xf, g4, w0, wr3, bias, bi, q0))
        outs.append(jnp.concatenate(chunks, axis=1))
    return jnp.stack(outs).reshape(b, co, h, w)


# speedup vs baseline: 1.2083x; 1.1775x over previous
import functools

import jax
import jax.numpy as jnp
from jax import lax
from jax.experimental import pallas as pl
from jax.experimental.pallas import tpu as pltpu
from jax.experimental.pallas import tpu_sc as plsc

_K = 9
_NBR = _K - 1
_TQ = 512
_TR = 512
_CP = 128
_NCHUNKQ = 2


def _table_body(qt_ref, tab_ref):
    qt = qt_ref[0]
    tab_ref[0] = jnp.concatenate(
        [qt.T, jnp.zeros((qt.shape[1], _CP - qt.shape[0]), jnp.float32)],
        axis=1)


def _table_call(xf):
    b, c, n = xf.shape
    return pl.pallas_call(
        _table_body,
        grid=(b, n // _TQ),
        in_specs=[pl.BlockSpec((1, c, _TQ), lambda i, q: (i, 0, q))],
        out_specs=pl.BlockSpec((1, _TQ, _CP), lambda i, q: (i, q, 0)),
        out_shape=jax.ShapeDtypeStruct((b, n, _CP), jnp.float32),
    )(xf)


def _topk_body(qt_ref, xf_ref, idx_ref, *, q0):
    qi = pl.program_id(0)
    qt = qt_ref[0]
    xf = xf_ref[0]
    c, n = xf.shape
    dot = lax.dot_general(qt, xf, (((0,), (0,)), ((), ())),
                          preferred_element_type=jnp.float32)
    nk = jnp.sum(xf * xf, axis=0, keepdims=True)
    r = nk - 2.0 * dot
    lane = lax.broadcasted_iota(jnp.int32, r.shape, 1)
    row = q0 + qi * _TQ + lax.broadcasted_iota(jnp.int32, r.shape, 0)
    inf = jnp.float32(jnp.inf)
    r = jnp.where(lane == row, inf, r)
    lane_f = lane.astype(jnp.float32)
    lane8 = lax.broadcasted_iota(jnp.int32, (r.shape[0], _NBR), 1)
    acc = jnp.zeros((r.shape[0], _NBR), jnp.float32)
    for k in range(_NBR):
        m = jnp.min(r, axis=1, keepdims=True)
        i_f = jnp.min(jnp.where(r == m, lane_f, inf), axis=1,
                      keepdims=True)
        acc = jnp.where(lane8 == k, i_f, acc)
        r = jnp.where(lane_f == i_f, inf, r)
    idx_ref[0] = acc.T.astype(jnp.int32)


def _topk_call(xf, b, q0, nq):
    _, c, n = xf.shape
    t0 = q0 // _TQ
    return pl.pallas_call(
        functools.partial(_topk_body, q0=q0),
        grid=(nq // _TQ,),
        in_specs=[
            pl.BlockSpec((1, c, _TQ), lambda q: (b, 0, t0 + q)),
            pl.BlockSpec((1, c, n), lambda q: (b, 0, 0)),
        ],
        out_specs=pl.BlockSpec((1, _NBR, _TQ), lambda q: (q, 0, 0)),
        out_shape=jax.ShapeDtypeStruct((nq // _TQ, _NBR, _TQ), jnp.int32),
    )(xf, xf)


def _make_gather(tot, cp):
    info = plsc.get_sparse_core_info()
    nw = info.num_cores * info.num_subcores
    per_w = tot // nw
    nchunk = per_w // 128
    nbuf = min(6, nchunk)
    ahead = max(1, nbuf - 2)
    mesh = plsc.VectorSubcoreMesh(core_axis_name="c", subcore_axis_name="s")

    @functools.partial(
        pl.kernel, mesh=mesh,
        out_type=jax.ShapeDtypeStruct((tot, cp), jnp.float32),
        scratch_types=[
            pltpu.VMEM((nchunk, 128), jnp.int32),
            pltpu.VMEM((nbuf, 128, cp), jnp.float32),
            pltpu.SemaphoreType.DMA,
            pltpu.SemaphoreType.DMA,
        ],
    )
    def gath(table_hbm, idx_hbm, out_hbm, idx_v, rows_v, gsem, wsem):
        wid = lax.axis_index("s") * info.num_cores + lax.axis_index("c")
        base = wid * per_w
        pltpu.sync_copy(idx_hbm.at[pl.ds(wid * nchunk, nchunk)], idx_v)
        gd = [None] * nchunk
        wd = [None] * nchunk
        for ck in range(min(ahead, nchunk)):
            gd[ck] = pltpu.async_copy(
                table_hbm.at[idx_v.at[ck]], rows_v.at[ck % nbuf], gsem)
        for ck in range(nchunk):
            nx = ck + ahead
            if nx < nchunk:
                if nx >= nbuf:
                    wd[nx - nbuf].wait()
                gd[nx] = pltpu.async_copy(
                    table_hbm.at[idx_v.at[nx]], rows_v.at[nx % nbuf], gsem)
            gd[ck].wait()
            wd[ck] = pltpu.async_copy(
                rows_v.at[ck % nbuf],
                out_hbm.at[pl.ds(base + ck * 128, 128)], wsem)
        for ck in range(max(0, nchunk - nbuf), nchunk):
            wd[ck].wait()

    return gath


def _conv_body(xfq_ref, g_ref, w0_ref, wr_ref, bias_ref, out_ref):
    c = w0_ref.shape[0]
    acc = lax.dot_general(xfq_ref[0], w0_ref[...], (((0,), (0,)), ((), ())),
                          preferred_element_type=jnp.float32)
    acc += bias_ref[...]
    for k in range(_NBR):
        acc += lax.dot_general(g_ref[0, k][:, :c], wr_ref[k],
                               (((1,), (0,)), ((), ())),
                               preferred_element_type=jnp.float32)
    out_ref[...] = acc.T


def _conv_call(xf, g4, w0, wr3, bias, b, q0):
    _, c, n = xf.shape
    nq = g4.shape[0] * g4.shape[2]
    co = w0.shape[1]
    t0 = q0 // _TR
    return pl.pallas_call(
        _conv_body,
        grid=(nq // _TR,),
        in_specs=[
            pl.BlockSpec((1, c, _TR), lambda t: (b, 0, t0 + t)),
            pl.BlockSpec((1, _NBR, _TR, _CP), lambda t: (t, 0, 0, 0)),
            pl.BlockSpec((c, co), lambda t: (0, 0)),
            pl.BlockSpec((_NBR, c, co), lambda t: (0, 0, 0)),
            pl.BlockSpec((1, co), lambda t: (0, 0)),
        ],
        out_specs=pl.BlockSpec((co, _TR), lambda t: (0, t)),
        out_shape=jax.ShapeDtypeStruct((co, nq), jnp.float32),
    )(xf, g4, w0, wr3, bias)


def kernel(x, conv_w, conv_b):
    b, c, h, w = x.shape
    n = h * w
    nq = n // _NCHUNKQ
    co = conv_w.shape[0]
    xf = x.reshape(b, c, n)

    wt = jnp.transpose(conv_w, (2, 1, 0))
    w0 = wt[0]
    wr3 = wt[1:]
    bias = conv_b.reshape(1, co)

    tables = _table_call(xf)
    gath = _make_gather(nq * _NBR, _CP)

    outs = []
    for bi in range(b):
        chunks = []
        for ci in range(_NCHUNKQ):
            q0 = ci * nq
            idx = _topk_call(xf, bi, q0, nq)
            flat_idx = idx.reshape(nq * _NBR // 128, 128)
            g = gath(tables[bi], flat_idx)
            g4 = g.reshape(nq // _TQ, _NBR, _TQ, _CP)
            chunks.append(_conv_call(xf, g4, w0, wr3, bias, bi, q0))
        outs.append(jnp.concatenate(chunks, axis=1))
    return jnp.stack(outs).reshape(b, co, h, w)
